# baseline (speedup 1.0000x reference)
import functools

import jax
import jax.numpy as jnp
from jax import lax
from jax.experimental import pallas as pl
from jax.experimental.pallas import tpu as pltpu
from jax.experimental.pallas import tpu_sc as plsc

VOCAB = 1000
VPAD = 1024
B, T = 1024, 50
N = B * T
NC, NS = 2, 16
NW = NC * NS
PER_W = N // NW
PER_B = B // NW
LG = VPAD // 128


def _mesh():
    return plsc.VectorSubcoreMesh(
        core_axis_name="c", subcore_axis_name="s", num_cores=NC, num_subcores=NS
    )


@functools.partial(
    pl.kernel,
    out_type=jax.ShapeDtypeStruct((B, T, VPAD), jnp.float32),
    mesh=_mesh(),
    scratch_types=[
        pltpu.VMEM((PER_W,), jnp.int32),
        pltpu.VMEM((LG * PER_B * 48,), jnp.int32),
        pltpu.VMEM((1, T, VPAD), jnp.float32),
    ]
    + [pltpu.VMEM((16, 128), jnp.float32) for _ in range(LG)]
    + [pltpu.SemaphoreType.DMA],
)
def _gather_kernel(idx_hbm, t8_hbm, out_hbm, idx_v, pidx_v, acc, *rest):
    pieces = rest[:LG]
    gsem = rest[LG]
    cid = lax.axis_index("c")
    sid = lax.axis_index("s")
    wid = sid * NC + cid
    base_t = wid * PER_W
    base_b = wid * PER_B

    pltpu.sync_copy(idx_hbm.at[pl.ds(base_t, PER_W)], idx_v)

    def pre_body(b, _):
        for t0 in (0, 16, 32):
            v = idx_v[pl.ds(50 * b + t0, 16)]
            p_base = (v >> 3) * 64 + (v & 7)
            for l in range(LG):
                pidx_v[pl.ds(1536 * l + 48 * b + t0, 16)] = p_base + l * 8
        return 0

    lax.fori_loop(0, PER_B, pre_body, 0)

    def batch_body(b, _):
        copies = [
            pltpu.async_copy(
                t8_hbm.at[pidx_v.at[pl.ds(1536 * l + 48 * b, 48)]],
                acc.at[0, pl.ds(0, 48), pl.ds(128 * l, 128)],
                gsem,
            )
            for l in range(LG)
        ]
        v = idx_v[pl.ds(50 * b + 34, 16)]
        p_base = (v >> 3) * 64 + (v & 7)
        copies += [
            pltpu.async_copy(t8_hbm.at[p_base + l * 8], pieces[l], gsem)
            for l in range(LG)
        ]
        for cp in copies:
            cp.wait()
        for j in (14, 15):
            t = 34 + j
            for l in range(LG):
                for k in range(8):
                    acc[0, t, pl.ds(128 * l + 16 * k, 16)] = pieces[l][
                        j, pl.ds(16 * k, 16)
                    ]
        pltpu.sync_copy(acc, out_hbm.at[pl.ds(base_b + b, 1)])
        return 0

    lax.fori_loop(0, PER_B, batch_body, 0)


def kernel(idx, table):
    flat_idx = idx.reshape(N).astype(jnp.int32)
    table_p = jnp.pad(table, ((0, 0), (0, VPAD - VOCAB)))
    t8 = (
        table_p.reshape(125, 8, 8, 128)
        .transpose(0, 2, 1, 3)
        .reshape(VOCAB * 8, 128)
    )
    out = _gather_kernel(flat_idx, t8)
    return out[:, :, :VOCAB]


# speedup vs baseline: 1.0022x; 1.0022x over previous
import functools

import jax
import jax.numpy as jnp
from jax import lax
from jax.experimental import pallas as pl
from jax.experimental.pallas import tpu as pltpu
from jax.experimental.pallas import tpu_sc as plsc

VOCAB = 1000
VPAD = 1024
B, T = 1024, 50
N = B * T
NC, NS = 2, 16
NW = NC * NS
PER_W = N // NW
PER_B = B // NW
LG = VPAD // 128


def _mesh():
    return plsc.VectorSubcoreMesh(
        core_axis_name="c", subcore_axis_name="s", num_cores=NC, num_subcores=NS
    )


@functools.partial(
    pl.kernel,
    out_type=jax.ShapeDtypeStruct((B, T, VPAD), jnp.float32),
    mesh=_mesh(),
    scratch_types=[
        pltpu.VMEM((PER_W,), jnp.int32),
        pltpu.VMEM((1, T, VPAD), jnp.float32),
    ]
    + [pltpu.VMEM((16, 128), jnp.float32) for _ in range(LG)]
    + [pltpu.SemaphoreType.DMA],
)
def _gather_kernel(idx_hbm, t8_hbm, out_hbm, idx_v, acc, *rest):
    pieces = rest[:LG]
    gsem = rest[LG]
    cid = lax.axis_index("c")
    sid = lax.axis_index("s")
    wid = sid * NC + cid
    base_t = wid * PER_W
    base_b = wid * PER_B

    pltpu.sync_copy(idx_hbm.at[pl.ds(base_t, PER_W)], idx_v)

    def batch_body(b, _):
        copies = []
        for t0, tail in ((0, False), (16, False), (32, False), (34, True)):
            v = idx_v[pl.ds(50 * b + t0, 16)]
            p_base = (v >> 3) * 64 + (v & 7)
            if not tail:
                copies += [
                    pltpu.async_copy(
                        t8_hbm.at[p_base + l * 8],
                        acc.at[0, pl.ds(t0, 16), pl.ds(128 * l, 128)],
                        gsem,
                    )
                    for l in range(LG)
                ]
            else:
                copies += [
                    pltpu.async_copy(t8_hbm.at[p_base + l * 8], pieces[l], gsem)
                    for l in range(LG)
                ]
        for cp in copies:
            cp.wait()
        for j in (14, 15):
            t = 34 + j
            for l in range(LG):
                for k in range(8):
                    acc[0, t, pl.ds(128 * l + 16 * k, 16)] = pieces[l][
                        j, pl.ds(16 * k, 16)
                    ]
        pltpu.sync_copy(acc, out_hbm.at[pl.ds(base_b + b, 1)])
        return 0

    lax.fori_loop(0, PER_B, batch_body, 0)


def kernel(idx, table):
    flat_idx = idx.reshape(N).astype(jnp.int32)
    table_p = jnp.pad(table, ((0, 0), (0, VPAD - VOCAB)))
    t8 = (
        table_p.reshape(125, 8, 8, 128)
        .transpose(0, 2, 1, 3)
        .reshape(VOCAB * 8, 128)
    )
    out = _gather_kernel(flat_idx, t8)
    return out[:, :, :VOCAB]
